# baseline (speedup 1.0000x reference)
import functools

import jax
import jax.numpy as jnp
from jax import lax
from jax.experimental import pallas as pl
from jax.experimental.pallas import tpu as pltpu
from jax.experimental.pallas import tpu_sc as plsc

V = 100000
E = 200
EP = 256
H = 128
B = 4096
L = 50

_NW = 32
_CHUNK = 128


def _sc_gather(seq_flat, table_p):
    n = seq_flat.shape[0]
    per_w = n // _NW
    chunks = per_w // _CHUNK
    mesh = plsc.VectorSubcoreMesh(core_axis_name="c", subcore_axis_name="s")

    @functools.partial(
        pl.kernel,
        out_type=jax.ShapeDtypeStruct((n, 128), jnp.float32),
        mesh=mesh,
        scratch_types=[
            pltpu.VMEM((_CHUNK,), jnp.int32),
            pltpu.VMEM((_CHUNK, 128), jnp.float32),
            pltpu.SemaphoreType.DMA,
        ],
    )
    def gather_kernel(seq_hbm, table_hbm, out_hbm, idx_v, rows_v, sem):
        wid = lax.axis_index("s") * 2 + lax.axis_index("c")
        base = wid * per_w

        def body(g, carry):
            off = base + g * _CHUNK
            pltpu.sync_copy(seq_hbm.at[pl.ds(off, _CHUNK)], idx_v)
            pltpu.async_copy(table_hbm.at[idx_v], rows_v, sem).wait()
            pltpu.sync_copy(rows_v, out_hbm.at[pl.ds(off, _CHUNK)])
            return carry

        lax.fori_loop(0, chunks, body, 0)

    return gather_kernel(seq_flat, table_p)


_PAD_BV = 5000


def _rne16(f):
    u = lax.bitcast_convert_type(f, jnp.uint32)
    return (u + jnp.uint32(0x7FFF) + ((u >> 16) & jnp.uint32(1))) >> 16


def _pad_body(t_ref, o_ref):
    x = t_ref[...]
    lo = x[:, :128]
    hi = jnp.concatenate(
        [x[:, 128:E], jnp.zeros((_PAD_BV, EP - E), jnp.float32)], axis=1
    )
    w = _rne16(lo) | (_rne16(hi) << 16)
    o_ref[...] = lax.bitcast_convert_type(w, jnp.float32)


def _pad_table(table):
    return pl.pallas_call(
        _pad_body,
        grid=(V // _PAD_BV,),
        in_specs=[pl.BlockSpec((_PAD_BV, E), lambda i: (i, 0))],
        out_specs=pl.BlockSpec((_PAD_BV, 128), lambda i: (i, 0)),
        out_shape=jax.ShapeDtypeStruct((V, 128), jnp.float32),
        compiler_params=pltpu.CompilerParams(
            dimension_semantics=("arbitrary",),
        ),
    )(table)


def _unpack_x(x_ref):
    xw = lax.bitcast_convert_type(x_ref[0], jnp.uint32)
    return jnp.concatenate(
        [
            lax.bitcast_convert_type(xw << 16, jnp.float32),
            lax.bitcast_convert_type(xw & jnp.uint32(0xFFFF0000), jnp.float32),
        ],
        axis=1,
    ).astype(jnp.bfloat16)


def _sigmoid_t(z):
    return 0.5 * jnp.tanh(0.5 * z) + 0.5


_UNROLL = 5


def _lstm_body(x_ref, wx_ref, wh_ref, b_ref, hin_ref, cin_ref,
               h_out, c_out, h_s, c_s):
    t = pl.program_id(1)
    nt = pl.num_programs(1)

    @pl.when(t == 0)
    def _init():
        h_s[...] = hin_ref[...]
        c_s[...] = cin_ref[...]

    h = h_s[...]
    c = c_s[...]
    bias = b_ref[...]
    for tt in range(_UNROLL):
        xw = lax.bitcast_convert_type(x_ref[tt], jnp.uint32)
        x = jnp.concatenate(
            [
                lax.bitcast_convert_type(xw << 16, jnp.float32),
                lax.bitcast_convert_type(xw & jnp.uint32(0xFFFF0000),
                                         jnp.float32),
            ],
            axis=1,
        ).astype(jnp.bfloat16)
        gates = (
            jnp.dot(x, wx_ref[...], preferred_element_type=jnp.float32)
            + jnp.dot(h.astype(jnp.bfloat16), wh_ref[...],
                      preferred_element_type=jnp.float32)
            + bias
        )
        i = _sigmoid_t(gates[:, 0 * H:1 * H])
        f = _sigmoid_t(gates[:, 1 * H:2 * H])
        g = jnp.tanh(gates[:, 2 * H:3 * H])
        o = _sigmoid_t(gates[:, 3 * H:4 * H])
        c = f * c + i * g
        h = o * jnp.tanh(c)
    h_s[...] = h
    c_s[...] = c

    @pl.when(t == nt - 1)
    def _emit():
        h_out[...] = h
        c_out[...] = c


def _lstm(xs, wx, wh, bias, h_in, c_in, block_b):
    nb = B // block_b
    lc = xs.shape[0]
    nt = lc // _UNROLL
    return pl.pallas_call(
        _lstm_body,
        grid=(nb, nt),
        in_specs=[
            pl.BlockSpec((_UNROLL, block_b, 128), lambda b, t: (t, b, 0)),
            pl.BlockSpec((EP, 4 * H), lambda b, t: (0, 0)),
            pl.BlockSpec((H, 4 * H), lambda b, t: (0, 0)),
            pl.BlockSpec((1, 4 * H), lambda b, t: (0, 0)),
            pl.BlockSpec((block_b, H), lambda b, t: (b, 0)),
            pl.BlockSpec((block_b, H), lambda b, t: (b, 0)),
        ],
        out_specs=[
            pl.BlockSpec((block_b, H), lambda b, t: (b, 0)),
            pl.BlockSpec((block_b, H), lambda b, t: (b, 0)),
        ],
        out_shape=[
            jax.ShapeDtypeStruct((B, H), jnp.float32),
            jax.ShapeDtypeStruct((B, H), jnp.float32),
        ],
        scratch_shapes=[
            pltpu.VMEM((block_b, H), jnp.float32),
            pltpu.VMEM((block_b, H), jnp.float32),
        ],
        compiler_params=pltpu.CompilerParams(
            dimension_semantics=("arbitrary", "arbitrary"),
        ),
    )(xs, wx, wh, bias, h_in, c_in)


_NCH = 2
_LC = L // _NCH


def kernel(sequence, table, W_ih, W_hh, b_ih, b_hh):
    seq_t = jnp.transpose(sequence, (1, 0)).reshape(-1).astype(jnp.int32)
    table_p = _pad_table(table)
    wx = jnp.pad(jnp.transpose(W_ih, (1, 0)), ((0, EP - E), (0, 0))).astype(jnp.bfloat16)
    wh = jnp.transpose(W_hh, (1, 0)).astype(jnp.bfloat16)
    bias = (b_ih + b_hh).reshape(1, 4 * H)
    h = jnp.zeros((B, H), jnp.float32)
    c = jnp.zeros((B, H), jnp.float32)
    nseg = _LC * B
    xs_prev = _sc_gather(seq_t[:nseg], table_p).reshape(_LC, B, 128)
    for k in range(_NCH):
        if k + 1 < _NCH:
            xs_next = _sc_gather(
                seq_t[(k + 1) * nseg:(k + 2) * nseg], table_p
            ).reshape(_LC, B, 128)
        h, c = _lstm(xs_prev, wx, wh, bias, h, c, block_b=2048)
        if k + 1 < _NCH:
            xs_prev = xs_next
    return (h[None], c[None])


# speedup vs baseline: 1.0577x; 1.0577x over previous
import functools

import jax
import jax.numpy as jnp
from jax import lax
from jax.experimental import pallas as pl
from jax.experimental.pallas import tpu as pltpu
from jax.experimental.pallas import tpu_sc as plsc

V = 100000
E = 200
EP = 256
H = 128
B = 4096
L = 50

_NW = 32
_CHUNK = 128


def _sc_gather(seq_flat, table_p):
    n = seq_flat.shape[0]
    per_w = n // _NW
    chunks = per_w // _CHUNK
    mesh = plsc.VectorSubcoreMesh(core_axis_name="c", subcore_axis_name="s")

    @functools.partial(
        pl.kernel,
        out_type=jax.ShapeDtypeStruct((n, 128), jnp.float32),
        mesh=mesh,
        scratch_types=[
            pltpu.VMEM((_CHUNK,), jnp.int32),
            pltpu.VMEM((_CHUNK, 128), jnp.float32),
            pltpu.SemaphoreType.DMA,
        ],
    )
    def gather_kernel(seq_hbm, table_hbm, out_hbm, idx_v, rows_v, sem):
        wid = lax.axis_index("s") * 2 + lax.axis_index("c")
        base = wid * per_w

        def body(g, carry):
            off = base + g * _CHUNK
            pltpu.sync_copy(seq_hbm.at[pl.ds(off, _CHUNK)], idx_v)
            pltpu.async_copy(table_hbm.at[idx_v], rows_v, sem).wait()
            pltpu.sync_copy(rows_v, out_hbm.at[pl.ds(off, _CHUNK)])
            return carry

        lax.fori_loop(0, chunks, body, 0)

    return gather_kernel(seq_flat, table_p)


_PAD_BV = 5000


def _rne16(f):
    u = lax.bitcast_convert_type(f, jnp.uint32)
    return (u + jnp.uint32(0x7FFF) + ((u >> 16) & jnp.uint32(1))) >> 16


def _pad_body(t_ref, o_ref):
    x = t_ref[...]
    lo = x[:, :128]
    hi = jnp.concatenate(
        [x[:, 128:E], jnp.zeros((_PAD_BV, EP - E), jnp.float32)], axis=1
    )
    w = _rne16(lo) | (_rne16(hi) << 16)
    o_ref[...] = lax.bitcast_convert_type(w, jnp.float32)


def _pad_table(table):
    return pl.pallas_call(
        _pad_body,
        grid=(V // _PAD_BV,),
        in_specs=[pl.BlockSpec((_PAD_BV, E), lambda i: (i, 0))],
        out_specs=pl.BlockSpec((_PAD_BV, 128), lambda i: (i, 0)),
        out_shape=jax.ShapeDtypeStruct((V, 128), jnp.float32),
        compiler_params=pltpu.CompilerParams(
            dimension_semantics=("arbitrary",),
        ),
    )(table)


def _unpack_x(x_ref):
    xw = lax.bitcast_convert_type(x_ref[0], jnp.uint32)
    return jnp.concatenate(
        [
            lax.bitcast_convert_type(xw << 16, jnp.float32),
            lax.bitcast_convert_type(xw & jnp.uint32(0xFFFF0000), jnp.float32),
        ],
        axis=1,
    ).astype(jnp.bfloat16)


def _sigmoid_t(z):
    return 0.5 * jnp.tanh(0.5 * z) + 0.5


_UNROLL = 5


def _lstm_body(x_ref, wx_ref, wh_ref, b_ref, hin_ref, cin_ref,
               h_out, c_out, h_s, c_s):
    t = pl.program_id(1)
    nt = pl.num_programs(1)

    @pl.when(t == 0)
    def _init():
        h_s[...] = hin_ref[...]
        c_s[...] = cin_ref[...]

    h = h_s[...]
    c = c_s[...]
    bias = b_ref[...]
    for tt in range(_UNROLL):
        xw = lax.bitcast_convert_type(x_ref[tt], jnp.uint32)
        x = jnp.concatenate(
            [
                lax.bitcast_convert_type(xw << 16, jnp.float32),
                lax.bitcast_convert_type(xw & jnp.uint32(0xFFFF0000),
                                         jnp.float32),
            ],
            axis=1,
        ).astype(jnp.bfloat16)
        gates = (
            jnp.dot(x, wx_ref[...], preferred_element_type=jnp.float32)
            + jnp.dot(h.astype(jnp.bfloat16), wh_ref[...],
                      preferred_element_type=jnp.float32)
            + bias
        )
        i = _sigmoid_t(gates[:, 0 * H:1 * H])
        f = _sigmoid_t(gates[:, 1 * H:2 * H])
        g = jnp.tanh(gates[:, 2 * H:3 * H])
        o = _sigmoid_t(gates[:, 3 * H:4 * H])
        c = f * c + i * g
        h = o * jnp.tanh(c)
    h_s[...] = h
    c_s[...] = c

    @pl.when(t == nt - 1)
    def _emit():
        h_out[...] = h
        c_out[...] = c


def _lstm(xs, wx, wh, bias, h_in, c_in, block_b):
    nb = B // block_b
    lc = xs.shape[0]
    nt = lc // _UNROLL
    return pl.pallas_call(
        _lstm_body,
        grid=(nb, nt),
        in_specs=[
            pl.BlockSpec((_UNROLL, block_b, 128), lambda b, t: (t, b, 0)),
            pl.BlockSpec((EP, 4 * H), lambda b, t: (0, 0)),
            pl.BlockSpec((H, 4 * H), lambda b, t: (0, 0)),
            pl.BlockSpec((1, 4 * H), lambda b, t: (0, 0)),
            pl.BlockSpec((block_b, H), lambda b, t: (b, 0)),
            pl.BlockSpec((block_b, H), lambda b, t: (b, 0)),
        ],
        out_specs=[
            pl.BlockSpec((block_b, H), lambda b, t: (b, 0)),
            pl.BlockSpec((block_b, H), lambda b, t: (b, 0)),
        ],
        out_shape=[
            jax.ShapeDtypeStruct((B, H), jnp.float32),
            jax.ShapeDtypeStruct((B, H), jnp.float32),
        ],
        scratch_shapes=[
            pltpu.VMEM((block_b, H), jnp.float32),
            pltpu.VMEM((block_b, H), jnp.float32),
        ],
        compiler_params=pltpu.CompilerParams(
            dimension_semantics=("arbitrary", "arbitrary"),
        ),
    )(xs, wx, wh, bias, h_in, c_in)


_NCH = 5
_LC = L // _NCH


def kernel(sequence, table, W_ih, W_hh, b_ih, b_hh):
    seq_t = jnp.transpose(sequence, (1, 0)).reshape(-1).astype(jnp.int32)
    table_p = _pad_table(table)
    wx = jnp.pad(jnp.transpose(W_ih, (1, 0)), ((0, EP - E), (0, 0))).astype(jnp.bfloat16)
    wh = jnp.transpose(W_hh, (1, 0)).astype(jnp.bfloat16)
    bias = (b_ih + b_hh).reshape(1, 4 * H)
    h = jnp.zeros((B, H), jnp.float32)
    c = jnp.zeros((B, H), jnp.float32)
    nseg = _LC * B
    xs_prev = _sc_gather(seq_t[:nseg], table_p).reshape(_LC, B, 128)
    for k in range(_NCH):
        if k + 1 < _NCH:
            xs_next = _sc_gather(
                seq_t[(k + 1) * nseg:(k + 2) * nseg], table_p
            ).reshape(_LC, B, 128)
        h, c = _lstm(xs_prev, wx, wh, bias, h, c, block_b=2048)
        if k + 1 < _NCH:
            xs_prev = xs_next
    return (h[None], c[None])


# speedup vs baseline: 1.0872x; 1.0279x over previous
import functools

import jax
import jax.numpy as jnp
from jax import lax
from jax.experimental import pallas as pl
from jax.experimental.pallas import tpu as pltpu
from jax.experimental.pallas import tpu_sc as plsc

V = 100000
E = 200
EP = 256
H = 128
B = 4096
L = 50

_NW = 32
_CHUNK = 128


def _sc_gather(seq_flat, table_p):
    n = seq_flat.shape[0]
    per_w = n // _NW
    chunks = per_w // _CHUNK
    mesh = plsc.VectorSubcoreMesh(core_axis_name="c", subcore_axis_name="s")

    @functools.partial(
        pl.kernel,
        out_type=jax.ShapeDtypeStruct((n, 128), jnp.float32),
        mesh=mesh,
        scratch_types=[
            pltpu.VMEM((_CHUNK,), jnp.int32),
            pltpu.VMEM((_CHUNK, 128), jnp.float32),
            pltpu.SemaphoreType.DMA,
        ],
    )
    def gather_kernel(seq_hbm, table_hbm, out_hbm, idx_v, rows_v, sem):
        wid = lax.axis_index("s") * 2 + lax.axis_index("c")
        base = wid * per_w

        def body(g, carry):
            off = base + g * _CHUNK
            pltpu.sync_copy(seq_hbm.at[pl.ds(off, _CHUNK)], idx_v)
            pltpu.async_copy(table_hbm.at[idx_v], rows_v, sem).wait()
            pltpu.sync_copy(rows_v, out_hbm.at[pl.ds(off, _CHUNK)])
            return carry

        lax.fori_loop(0, chunks, body, 0)

    return gather_kernel(seq_flat, table_p)


_PAD_BV = 5000


def _rne16(f):
    u = lax.bitcast_convert_type(f, jnp.uint32)
    return (u + jnp.uint32(0x7FFF) + ((u >> 16) & jnp.uint32(1))) >> 16


def _pad_body(t_ref, o_ref):
    x = t_ref[...]
    lo = x[:, :128]
    hi = jnp.concatenate(
        [x[:, 128:E], jnp.zeros((_PAD_BV, EP - E), jnp.float32)], axis=1
    )
    w = _rne16(lo) | (_rne16(hi) << 16)
    o_ref[...] = lax.bitcast_convert_type(w, jnp.float32)


def _pad_table(table):
    return pl.pallas_call(
        _pad_body,
        grid=(V // _PAD_BV,),
        in_specs=[pl.BlockSpec((_PAD_BV, E), lambda i: (i, 0))],
        out_specs=pl.BlockSpec((_PAD_BV, 128), lambda i: (i, 0)),
        out_shape=jax.ShapeDtypeStruct((V, 128), jnp.float32),
        compiler_params=pltpu.CompilerParams(
            dimension_semantics=("arbitrary",),
        ),
    )(table)


def _unpack_x(x_ref):
    xw = lax.bitcast_convert_type(x_ref[0], jnp.uint32)
    return jnp.concatenate(
        [
            lax.bitcast_convert_type(xw << 16, jnp.float32),
            lax.bitcast_convert_type(xw & jnp.uint32(0xFFFF0000), jnp.float32),
        ],
        axis=1,
    ).astype(jnp.bfloat16)


def _sigmoid_t(z):
    return 0.5 * jnp.tanh(0.5 * z) + 0.5


_UNROLL = 10


def _lstm_body(x_ref, wx_ref, wh_ref, b_ref, hin_ref, cin_ref,
               h_out, c_out, h_s, c_s):
    t = pl.program_id(1)
    nt = pl.num_programs(1)

    @pl.when(t == 0)
    def _init():
        h_s[...] = hin_ref[...]
        c_s[...] = cin_ref[...]

    h = h_s[...]
    c = c_s[...]
    bias = b_ref[...]
    for tt in range(_UNROLL):
        xw = lax.bitcast_convert_type(x_ref[tt], jnp.uint32)
        x = jnp.concatenate(
            [
                lax.bitcast_convert_type(xw << 16, jnp.float32),
                lax.bitcast_convert_type(xw & jnp.uint32(0xFFFF0000),
                                         jnp.float32),
            ],
            axis=1,
        ).astype(jnp.bfloat16)
        gates = (
            jnp.dot(x, wx_ref[...], preferred_element_type=jnp.float32)
            + jnp.dot(h.astype(jnp.bfloat16), wh_ref[...],
                      preferred_element_type=jnp.float32)
            + bias
        )
        i = _sigmoid_t(gates[:, 0 * H:1 * H])
        f = _sigmoid_t(gates[:, 1 * H:2 * H])
        g = jnp.tanh(gates[:, 2 * H:3 * H])
        o = _sigmoid_t(gates[:, 3 * H:4 * H])
        c = f * c + i * g
        h = o * jnp.tanh(c)
    h_s[...] = h
    c_s[...] = c

    @pl.when(t == nt - 1)
    def _emit():
        h_out[...] = h
        c_out[...] = c


def _lstm(xs, wx, wh, bias, h_in, c_in, block_b):
    nb = B // block_b
    lc = xs.shape[0]
    nt = lc // _UNROLL
    return pl.pallas_call(
        _lstm_body,
        grid=(nb, nt),
        in_specs=[
            pl.BlockSpec((_UNROLL, block_b, 128), lambda b, t: (t, b, 0)),
            pl.BlockSpec((EP, 4 * H), lambda b, t: (0, 0)),
            pl.BlockSpec((H, 4 * H), lambda b, t: (0, 0)),
            pl.BlockSpec((1, 4 * H), lambda b, t: (0, 0)),
            pl.BlockSpec((block_b, H), lambda b, t: (b, 0)),
            pl.BlockSpec((block_b, H), lambda b, t: (b, 0)),
        ],
        out_specs=[
            pl.BlockSpec((block_b, H), lambda b, t: (b, 0)),
            pl.BlockSpec((block_b, H), lambda b, t: (b, 0)),
        ],
        out_shape=[
            jax.ShapeDtypeStruct((B, H), jnp.float32),
            jax.ShapeDtypeStruct((B, H), jnp.float32),
        ],
        scratch_shapes=[
            pltpu.VMEM((block_b, H), jnp.float32),
            pltpu.VMEM((block_b, H), jnp.float32),
        ],
        compiler_params=pltpu.CompilerParams(
            dimension_semantics=("arbitrary", "arbitrary"),
        ),
    )(xs, wx, wh, bias, h_in, c_in)


_NCH = 5
_LC = L // _NCH


def kernel(sequence, table, W_ih, W_hh, b_ih, b_hh):
    seq_t = jnp.transpose(sequence, (1, 0)).reshape(-1).astype(jnp.int32)
    table_p = _pad_table(table)
    wx = jnp.pad(jnp.transpose(W_ih, (1, 0)), ((0, EP - E), (0, 0))).astype(jnp.bfloat16)
    wh = jnp.transpose(W_hh, (1, 0)).astype(jnp.bfloat16)
    bias = (b_ih + b_hh).reshape(1, 4 * H)
    h = jnp.zeros((B, H), jnp.float32)
    c = jnp.zeros((B, H), jnp.float32)
    nseg = _LC * B
    xs_prev = _sc_gather(seq_t[:nseg], table_p).reshape(_LC, B, 128)
    for k in range(_NCH):
        if k + 1 < _NCH:
            xs_next = _sc_gather(
                seq_t[(k + 1) * nseg:(k + 2) * nseg], table_p
            ).reshape(_LC, B, 128)
        h, c = _lstm(xs_prev, wx, wh, bias, h, c, block_b=1024)
        if k + 1 < _NCH:
            xs_prev = xs_next
    return (h[None], c[None])
